# baseline (speedup 1.0000x reference)
import functools

import jax
import jax.numpy as jnp
from jax.experimental import pallas as pl
from jax.experimental.pallas import tpu as pltpu

_B, _L, _H = 4, 2048, 4096
_G, _K = 4, 2
_HDG = _H // _G
_DM = _K * _HDG
_T = 512
_N = (_B * _L) // _T


def _top2(s0, s1, s2, s3):
    neg = jnp.float32(-jnp.inf)

    def top1(a0, a1, a2, a3):
        t01 = a1 > a0
        m01 = jnp.where(t01, a1, a0)
        i01 = jnp.where(t01, 1, 0)
        t23 = a3 > a2
        m23 = jnp.where(t23, a3, a2)
        i23 = jnp.where(t23, 3, 2)
        tf = m23 > m01
        return jnp.where(tf, m23, m01), jnp.where(tf, i23, i01)

    m_a, i_a = top1(s0, s1, s2, s3)
    s0b = jnp.where(i_a == 0, neg, s0)
    s1b = jnp.where(i_a == 1, neg, s1)
    s2b = jnp.where(i_a == 2, neg, s2)
    s3b = jnp.where(i_a == 3, neg, s3)
    m_b, i_b = top1(s0b, s1b, s2b, s3b)
    return m_a, i_a, m_b, i_b


def _coeffs(idx, gate):
    zero = jnp.float32(0)
    return [jnp.where(idx == g, gate, zero) for g in range(_G)]


def _mof_kernel(x_ref, wg_ref, wm_ref, out_ref, g_scratch, c_scratch):
    i = pl.program_id(0)
    cur = jax.lax.rem(i, 2)
    prev = jax.lax.rem(i + 1, 2)

    @pl.when(i < _N)
    def _stage_a():
        xb = x_ref[...]
        logits = jax.lax.dot_general(
            xb, wg_ref[...], (((1,), (1,)), ((), ())),
            preferred_element_type=jnp.float32)

        ls = [logits[:, g:g + 1] for g in range(4)]
        ld = [logits[:, 4 + g:5 + g] for g in range(4)]

        ms_a, is_a, ms_b, is_b = _top2(*ls)

        ca = _coeffs(is_a, jax.nn.sigmoid(ms_a))
        cb = _coeffs(is_b, jax.nn.sigmoid(ms_b))

        chunks = [xb[:, g * _HDG:(g + 1) * _HDG] for g in range(_G)]
        g_scratch[cur, :, :_HDG] = (ca[0] * chunks[0] + ca[1] * chunks[1]
                                    + ca[2] * chunks[2] + ca[3] * chunks[3])
        g_scratch[cur, :, _HDG:] = (cb[0] * chunks[0] + cb[1] * chunks[1]
                                    + cb[2] * chunks[2] + cb[3] * chunks[3])
        c_scratch[cur] = logits

    @pl.when(i > 0)
    def _stage_b():
        y = jax.lax.dot_general(
            g_scratch[prev], wm_ref[...], (((1,), (1,)), ((), ())),
            preferred_element_type=jnp.float32)
        ya = y[:, :_HDG]
        yb = y[:, _HDG:]
        cc = c_scratch[prev]
        ld = [cc[:, 4 + g:5 + g] for g in range(4)]
        md_a, id_a, md_b, id_b = _top2(*ld)
        da = _coeffs(id_a, jax.nn.sigmoid(md_a))
        db = _coeffs(id_b, jax.nn.sigmoid(md_b))
        for g in range(_G):
            out_ref[:, g * _HDG:(g + 1) * _HDG] = da[g] * ya + db[g] * yb


@functools.partial(jax.jit, static_argnames=())
def kernel(x, W_src, W_dst, W_model):
    b, l, h = x.shape
    n_tok = b * l
    xf = x.reshape(n_tok, h)
    wg = jnp.concatenate([W_src, W_dst], axis=0)
    out = pl.pallas_call(
        _mof_kernel,
        grid=(_N + 1,),
        in_specs=[
            pl.BlockSpec((_T, _H), lambda i: (jnp.minimum(i, _N - 1), 0)),
            pl.BlockSpec((2 * _G, _H), lambda i: (0, 0)),
            pl.BlockSpec((_DM, _DM), lambda i: (0, 0)),
        ],
        out_specs=pl.BlockSpec((_T, _H), lambda i: (jnp.maximum(i - 1, 0), 0)),
        out_shape=jax.ShapeDtypeStruct((n_tok, h), jnp.float32),
        scratch_shapes=[pltpu.VMEM((2, _T, _DM), jnp.float32),
                        pltpu.VMEM((2, _T, 2 * _G), jnp.float32)],
        compiler_params=pltpu.CompilerParams(
            vmem_limit_bytes=110 * 1024 * 1024),
    )(xf, wg, W_model)
    return out.reshape(b, l, h)


# speedup vs baseline: 1.0088x; 1.0088x over previous
import functools

import jax
import jax.numpy as jnp
from jax.experimental import pallas as pl
from jax.experimental.pallas import tpu as pltpu

_B, _L, _H = 4, 2048, 4096
_G, _K = 4, 2
_HDG = _H // _G
_DM = _K * _HDG
_T = 512
_N = (_B * _L) // _T


def _top2(s0, s1, s2, s3):
    neg = jnp.float32(-jnp.inf)

    def top1(a0, a1, a2, a3):
        t01 = a1 > a0
        m01 = jnp.where(t01, a1, a0)
        i01 = jnp.where(t01, 1, 0)
        t23 = a3 > a2
        m23 = jnp.where(t23, a3, a2)
        i23 = jnp.where(t23, 3, 2)
        tf = m23 > m01
        return jnp.where(tf, m23, m01), jnp.where(tf, i23, i01)

    m_a, i_a = top1(s0, s1, s2, s3)
    s0b = jnp.where(i_a == 0, neg, s0)
    s1b = jnp.where(i_a == 1, neg, s1)
    s2b = jnp.where(i_a == 2, neg, s2)
    s3b = jnp.where(i_a == 3, neg, s3)
    m_b, i_b = top1(s0b, s1b, s2b, s3b)
    return m_a, i_a, m_b, i_b


def _coeffs(idx, gate):
    zero = jnp.float32(0)
    return [jnp.where(idx == g, gate, zero) for g in range(_G)]


def _mof_kernel(x_ref, wg_ref, wm_ref, out_ref, g_scratch, c_scratch):
    i = pl.program_id(0)
    cur = jax.lax.rem(i, 2)
    prev = jax.lax.rem(i + 1, 2)

    @pl.when(i < _N)
    def _stage_a():
        xb = x_ref[...]
        logits = jax.lax.dot_general(
            xb, wg_ref[...], (((1,), (1,)), ((), ())),
            preferred_element_type=jnp.float32)

        ls = [logits[:, g:g + 1] for g in range(4)]
        ld = [logits[:, 4 + g:5 + g] for g in range(4)]

        ms_a, is_a, ms_b, is_b = _top2(*ls)
        md_a, id_a, md_b, id_b = _top2(*ld)

        ca = _coeffs(is_a, jax.nn.sigmoid(ms_a))
        cb = _coeffs(is_b, jax.nn.sigmoid(ms_b))
        da = _coeffs(id_a, jax.nn.sigmoid(md_a))
        db = _coeffs(id_b, jax.nn.sigmoid(md_b))

        chunks = [xb[:, g * _HDG:(g + 1) * _HDG] for g in range(_G)]
        g_scratch[cur, :, :_HDG] = (ca[0] * chunks[0] + ca[1] * chunks[1]
                                    + ca[2] * chunks[2] + ca[3] * chunks[3])
        g_scratch[cur, :, _HDG:] = (cb[0] * chunks[0] + cb[1] * chunks[1]
                                    + cb[2] * chunks[2] + cb[3] * chunks[3])
        c_scratch[cur] = jnp.concatenate(da + db, axis=1)

    @pl.when(i > 0)
    def _stage_b():
        y = jax.lax.dot_general(
            g_scratch[prev], wm_ref[...], (((1,), (1,)), ((), ())),
            preferred_element_type=jnp.float32)
        ya = y[:, :_HDG]
        yb = y[:, _HDG:]
        cc = c_scratch[prev]
        for g in range(_G):
            out_ref[:, g * _HDG:(g + 1) * _HDG] = (
                cc[:, g:g + 1] * ya + cc[:, 4 + g:5 + g] * yb)


@functools.partial(jax.jit, static_argnames=())
def kernel(x, W_src, W_dst, W_model):
    b, l, h = x.shape
    n_tok = b * l
    xf = x.reshape(n_tok, h)
    wg = jnp.concatenate([W_src, W_dst], axis=0)
    out = pl.pallas_call(
        _mof_kernel,
        grid=(_N + 1,),
        in_specs=[
            pl.BlockSpec((_T, _H), lambda i: (jnp.minimum(i, _N - 1), 0)),
            pl.BlockSpec((2 * _G, _H), lambda i: (0, 0)),
            pl.BlockSpec((_DM, _DM), lambda i: (0, 0)),
        ],
        out_specs=pl.BlockSpec((_T, _H), lambda i: (jnp.maximum(i - 1, 0), 0)),
        out_shape=jax.ShapeDtypeStruct((n_tok, h), jnp.float32),
        scratch_shapes=[pltpu.VMEM((2, _T, _DM), jnp.float32),
                        pltpu.VMEM((2, _T, 2 * _G), jnp.float32)],
        compiler_params=pltpu.CompilerParams(
            vmem_limit_bytes=110 * 1024 * 1024),
    )(xf, wg, W_model)
    return out.reshape(b, l, h)


# speedup vs baseline: 1.0166x; 1.0078x over previous
import functools

import jax
import jax.numpy as jnp
from jax.experimental import pallas as pl
from jax.experimental.pallas import tpu as pltpu

_B, _L, _H = 4, 2048, 4096
_G, _K = 4, 2
_HDG = _H // _G
_DM = _K * _HDG
_T = 512
_N = (_B * _L) // _T


def _top2(s0, s1, s2, s3):
    neg = jnp.float32(-jnp.inf)

    def top1(a0, a1, a2, a3):
        t01 = a1 > a0
        m01 = jnp.where(t01, a1, a0)
        i01 = jnp.where(t01, 1, 0)
        t23 = a3 > a2
        m23 = jnp.where(t23, a3, a2)
        i23 = jnp.where(t23, 3, 2)
        tf = m23 > m01
        return jnp.where(tf, m23, m01), jnp.where(tf, i23, i01)

    m_a, i_a = top1(s0, s1, s2, s3)
    s0b = jnp.where(i_a == 0, neg, s0)
    s1b = jnp.where(i_a == 1, neg, s1)
    s2b = jnp.where(i_a == 2, neg, s2)
    s3b = jnp.where(i_a == 3, neg, s3)
    m_b, i_b = top1(s0b, s1b, s2b, s3b)
    return m_a, i_a, m_b, i_b


def _coeffs(idx, gate):
    zero = jnp.float32(0)
    return [jnp.where(idx == g, gate, zero) for g in range(_G)]


def _mof_kernel(x_ref, wg_ref, wm_ref, out_ref, g_scratch, c_scratch):
    i = pl.program_id(0)
    cur = jax.lax.rem(i, 2)
    prev = jax.lax.rem(i + 1, 2)

    @pl.when(i < _N)
    def _stage_a():
        xb = x_ref[...]
        logits_t = jax.lax.dot_general(
            xb, wg_ref[...], (((1,), (1,)), ((), ())),
            preferred_element_type=jnp.float32).T

        ls = [logits_t[g:g + 1, :] for g in range(4)]
        ld = [logits_t[4 + g:5 + g, :] for g in range(4)]

        ms_a, is_a, ms_b, is_b = _top2(*ls)
        md_a, id_a, md_b, id_b = _top2(*ld)

        ca = _coeffs(is_a, jax.nn.sigmoid(ms_a))
        cb = _coeffs(is_b, jax.nn.sigmoid(ms_b))
        da = _coeffs(id_a, jax.nn.sigmoid(md_a))
        db = _coeffs(id_b, jax.nn.sigmoid(md_b))

        cs = jnp.concatenate(ca + cb, axis=0).T
        chunks = [xb[:, g * _HDG:(g + 1) * _HDG] for g in range(_G)]
        g_scratch[cur, :, :_HDG] = (
            cs[:, 0:1] * chunks[0] + cs[:, 1:2] * chunks[1]
            + cs[:, 2:3] * chunks[2] + cs[:, 3:4] * chunks[3])
        g_scratch[cur, :, _HDG:] = (
            cs[:, 4:5] * chunks[0] + cs[:, 5:6] * chunks[1]
            + cs[:, 6:7] * chunks[2] + cs[:, 7:8] * chunks[3])
        c_scratch[cur] = jnp.concatenate(da + db, axis=0).T

    @pl.when(i > 0)
    def _stage_b():
        y = jax.lax.dot_general(
            g_scratch[prev], wm_ref[...], (((1,), (1,)), ((), ())),
            preferred_element_type=jnp.float32)
        ya = y[:, :_HDG]
        yb = y[:, _HDG:]
        cc = c_scratch[prev]
        for g in range(_G):
            out_ref[:, g * _HDG:(g + 1) * _HDG] = (
                cc[:, g:g + 1] * ya + cc[:, 4 + g:5 + g] * yb)


@functools.partial(jax.jit, static_argnames=())
def kernel(x, W_src, W_dst, W_model):
    b, l, h = x.shape
    n_tok = b * l
    xf = x.reshape(n_tok, h)
    wg = jnp.concatenate([W_src, W_dst], axis=0)
    out = pl.pallas_call(
        _mof_kernel,
        grid=(_N + 1,),
        in_specs=[
            pl.BlockSpec((_T, _H), lambda i: (jnp.minimum(i, _N - 1), 0)),
            pl.BlockSpec((2 * _G, _H), lambda i: (0, 0)),
            pl.BlockSpec((_DM, _DM), lambda i: (0, 0)),
        ],
        out_specs=pl.BlockSpec((_T, _H), lambda i: (jnp.maximum(i - 1, 0), 0)),
        out_shape=jax.ShapeDtypeStruct((n_tok, h), jnp.float32),
        scratch_shapes=[pltpu.VMEM((2, _T, _DM), jnp.float32),
                        pltpu.VMEM((2, _T, 2 * _G), jnp.float32)],
        compiler_params=pltpu.CompilerParams(
            vmem_limit_bytes=110 * 1024 * 1024),
    )(xf, wg, W_model)
    return out.reshape(b, l, h)
